# baseline (speedup 1.0000x reference)
import jax
import jax.numpy as jnp
from jax.experimental import pallas as pl
from jax.experimental.pallas import tpu as pltpu

B, N, C = 16, 20000, 91
K = 100
ROW_BLK = 2048
NB = (N + ROW_BLK - 1) // ROW_BLK
NROWS = NB * ROW_BLK // 128
NEG = float("-inf")
BIG = 2**30


def _select_rows_kernel(x_ref, ns_ref, rm_scratch):
    i = pl.program_id(1)
    rm = jnp.max(x_ref[...], axis=-1)
    n_of = jax.lax.broadcasted_iota(jnp.int32, (1, ROW_BLK), 1) + i * ROW_BLK
    rm = jnp.where(n_of < N, rm, NEG)
    rm_scratch[pl.ds(i * (ROW_BLK // 128), ROW_BLK // 128), :] = rm.reshape(
        ROW_BLK // 128, 128
    )

    @pl.when(i == NB - 1)
    def _():
        iota_r = jax.lax.broadcasted_iota(jnp.int32, (NROWS, 1), 0)
        iota_l = jax.lax.broadcasted_iota(jnp.int32, (1, 128), 1)
        rsum = jnp.max(rm_scratch[...], axis=1, keepdims=True)

        def body(it, carry):
            rsum, ns = carry
            m = jnp.max(rsum)
            rj = jnp.min(jnp.where(rsum == m, iota_r, BIG))
            row = rm_scratch[pl.ds(rj, 1), :]
            c = jnp.min(jnp.where(row == m, iota_l, BIG))
            n = rj * 128 + c
            new_row = jnp.where(iota_l == c, NEG, row)
            rm_scratch[pl.ds(rj, 1), :] = new_row
            rsum = jnp.where(iota_r == rj, jnp.max(new_row), rsum)
            ns = jnp.where(iota_l == it, n, ns)
            return rsum, ns

        _, ns = jax.lax.fori_loop(
            0, K, body, (rsum, jnp.zeros((1, 128), jnp.int32))
        )
        ns_ref[0, :, :] = ns


@jax.jit
def _select_rows(pred_logits):
    return pl.pallas_call(
        _select_rows_kernel,
        grid=(B, NB),
        in_specs=[pl.BlockSpec((1, ROW_BLK, C), lambda b, i: (b, i, 0))],
        out_specs=pl.BlockSpec((1, 1, 128), lambda b, i: (b, 0, 0)),
        out_shape=jax.ShapeDtypeStruct((B, 1, 128), jnp.int32),
        scratch_shapes=[pltpu.VMEM((NROWS, 128), jnp.float32)],
    )(pred_logits)


def _final_topk_kernel(x_ref, rid_ref, val_ref, lab_ref, ns_ref, d_scratch):
    iota_r = jax.lax.broadcasted_iota(jnp.int32, (K, 1), 0)
    iota_l = jax.lax.broadcasted_iota(jnp.int32, (1, 128), 1)
    d_scratch[...] = jnp.full((K, 128), NEG, jnp.float32)
    d_scratch[:, pl.ds(0, C)] = x_ref[0]
    rids = rid_ref[0]
    rsum = jnp.max(d_scratch[...], axis=1, keepdims=True)

    def body(it, carry):
        rsum, vals, labs, ns = carry
        m = jnp.max(rsum)
        rj = jnp.min(jnp.where(rsum == m, iota_r, BIG))
        row = d_scratch[pl.ds(rj, 1), :]
        c = jnp.min(jnp.where(row == m, iota_l, BIG))
        n = jnp.min(jnp.where(iota_l == rj, rids, BIG))
        new_row = jnp.where(iota_l == c, NEG, row)
        d_scratch[pl.ds(rj, 1), :] = new_row
        rsum = jnp.where(iota_r == rj, jnp.max(new_row), rsum)
        vals = jnp.where(iota_l == it, m, vals)
        labs = jnp.where(iota_l == it, c, labs)
        ns = jnp.where(iota_l == it, n, ns)
        return rsum, vals, labs, ns

    zf = jnp.zeros((1, 128), jnp.float32)
    zi = jnp.zeros((1, 128), jnp.int32)
    _, vals, labs, ns = jax.lax.fori_loop(0, K, body, (rsum, zf, zi, zi))
    val_ref[0, :, :] = jax.nn.sigmoid(vals)
    lab_ref[0, :, :] = labs
    ns_ref[0, :, :] = ns


@jax.jit
def _final_topk(cand, rids):
    return pl.pallas_call(
        _final_topk_kernel,
        grid=(B,),
        in_specs=[
            pl.BlockSpec((1, K, C), lambda b: (b, 0, 0)),
            pl.BlockSpec((1, 1, 128), lambda b: (b, 0, 0)),
        ],
        out_specs=[
            pl.BlockSpec((1, 1, 128), lambda b: (b, 0, 0)),
            pl.BlockSpec((1, 1, 128), lambda b: (b, 0, 0)),
            pl.BlockSpec((1, 1, 128), lambda b: (b, 0, 0)),
        ],
        out_shape=[
            jax.ShapeDtypeStruct((B, 1, 128), jnp.float32),
            jax.ShapeDtypeStruct((B, 1, 128), jnp.int32),
            jax.ShapeDtypeStruct((B, 1, 128), jnp.int32),
        ],
        scratch_shapes=[pltpu.VMEM((K, 128), jnp.float32)],
    )(cand, rids)


@jax.jit
def kernel(pred_logits, pred_boxes, target_sizes):
    rids = _select_rows(pred_logits)
    rids100 = jnp.sort(rids[:, 0, :K], axis=1)
    cand = jnp.take_along_axis(pred_logits, rids100[:, :, None], axis=1)
    rids_pad = jnp.pad(rids100, ((0, 0), (0, 128 - K)))[:, None, :]
    vals, labs, ns = _final_topk(cand, rids_pad)
    scores = vals[:, 0, :K]
    labels = labs[:, 0, :K]
    topk_boxes = ns[:, 0, :K]

    gather_idx = jnp.broadcast_to(topk_boxes[:, :, None], (B, K, 4))
    bx = jnp.take_along_axis(pred_boxes, gather_idx, axis=1)
    cx, cy, w, h = jnp.split(bx, 4, axis=-1)
    boxes = jnp.concatenate(
        [cx - 0.5 * w, cy - 0.5 * h, cx + 0.5 * w, cy + 0.5 * h], axis=-1
    )
    img_h = target_sizes[:, 0]
    img_w = target_sizes[:, 1]
    scale_fct = jnp.stack([img_w, img_h, img_w, img_h], axis=1)
    boxes = boxes * scale_fct[:, None, :]
    return scores, labels, boxes


# speedup vs baseline: 1.0364x; 1.0364x over previous
import jax
import jax.numpy as jnp
from jax.experimental import pallas as pl
from jax.experimental.pallas import tpu as pltpu

B, N, C = 16, 20000, 91
K = 100
ROW_BLK = 2048
NB = (N + ROW_BLK - 1) // ROW_BLK
NROWS = NB * ROW_BLK // 128
NEG = float("-inf")
BIG = 2**30


def _select_rows_kernel(x_ref, ns_ref, rm_scratch):
    i = pl.program_id(1)
    rm = jnp.max(x_ref[...], axis=-1)
    n_of = jax.lax.broadcasted_iota(jnp.int32, (1, ROW_BLK), 1) + i * ROW_BLK
    rm = jnp.where(n_of < N, rm, NEG)
    rm_scratch[pl.ds(i * (ROW_BLK // 128), ROW_BLK // 128), :] = rm.reshape(
        ROW_BLK // 128, 128
    )

    @pl.when(i == NB - 1)
    def _():
        iota_r = jax.lax.broadcasted_iota(jnp.int32, (NROWS, 1), 0)
        iota_l = jax.lax.broadcasted_iota(jnp.int32, (1, 128), 1)
        rsum = jnp.max(rm_scratch[...], axis=1, keepdims=True)

        def body(it, carry):
            rsum, ns = carry
            m = jnp.max(rsum)
            rj = jnp.min(jnp.where(rsum == m, iota_r, BIG))
            row = rm_scratch[pl.ds(rj, 1), :]
            c = jnp.min(jnp.where(row == m, iota_l, BIG))
            n = rj * 128 + c
            new_row = jnp.where(iota_l == c, NEG, row)
            rm_scratch[pl.ds(rj, 1), :] = new_row
            rsum = jnp.where(iota_r == rj, jnp.max(new_row), rsum)
            ns = jnp.where(iota_l == it, n, ns)
            return rsum, ns

        _, ns = jax.lax.fori_loop(
            0, K, body, (rsum, jnp.zeros((1, 128), jnp.int32))
        )
        ns_ref[0, :, :] = ns


@jax.jit
def _select_rows(pred_logits):
    return pl.pallas_call(
        _select_rows_kernel,
        grid=(B, NB),
        in_specs=[pl.BlockSpec((1, ROW_BLK, C), lambda b, i: (b, i, 0))],
        out_specs=pl.BlockSpec((1, 1, 128), lambda b, i: (b, 0, 0)),
        out_shape=jax.ShapeDtypeStruct((B, 1, 128), jnp.int32),
        scratch_shapes=[pltpu.VMEM((NROWS, 128), jnp.float32)],
    )(pred_logits)


def _final_topk_kernel(x_ref, rid_ref, val_ref, lab_ref, ns_ref, d_scratch):
    iota_r = jax.lax.broadcasted_iota(jnp.int32, (K, 1), 0)
    iota_l = jax.lax.broadcasted_iota(jnp.int32, (1, 128), 1)
    d_scratch[...] = jnp.full((K, 128), NEG, jnp.float32)
    d_scratch[:, pl.ds(0, C)] = x_ref[0]
    rids = rid_ref[0]
    rsum = jnp.max(d_scratch[...], axis=1, keepdims=True)

    def body(it, carry):
        rsum, vals, labs, ns = carry
        m = jnp.max(rsum)
        rj = jnp.min(jnp.where(rsum == m, iota_r, BIG))
        row = d_scratch[pl.ds(rj, 1), :]
        c = jnp.min(jnp.where(row == m, iota_l, BIG))
        n = jnp.min(jnp.where(iota_l == rj, rids, BIG))
        new_row = jnp.where(iota_l == c, NEG, row)
        d_scratch[pl.ds(rj, 1), :] = new_row
        rsum = jnp.where(iota_r == rj, jnp.max(new_row), rsum)
        vals = jnp.where(iota_l == it, m, vals)
        labs = jnp.where(iota_l == it, c, labs)
        ns = jnp.where(iota_l == it, n, ns)
        return rsum, vals, labs, ns

    zf = jnp.zeros((1, 128), jnp.float32)
    zi = jnp.zeros((1, 128), jnp.int32)
    _, vals, labs, ns = jax.lax.fori_loop(0, K, body, (rsum, zf, zi, zi))
    val_ref[0, :, :] = jax.nn.sigmoid(vals)
    lab_ref[0, :, :] = labs
    ns_ref[0, :, :] = ns


@jax.jit
def _final_topk(cand, rids):
    return pl.pallas_call(
        _final_topk_kernel,
        grid=(B,),
        in_specs=[
            pl.BlockSpec((1, K, C), lambda b: (b, 0, 0)),
            pl.BlockSpec((1, 1, 128), lambda b: (b, 0, 0)),
        ],
        out_specs=[
            pl.BlockSpec((1, 1, 128), lambda b: (b, 0, 0)),
            pl.BlockSpec((1, 1, 128), lambda b: (b, 0, 0)),
            pl.BlockSpec((1, 1, 128), lambda b: (b, 0, 0)),
        ],
        out_shape=[
            jax.ShapeDtypeStruct((B, 1, 128), jnp.float32),
            jax.ShapeDtypeStruct((B, 1, 128), jnp.int32),
            jax.ShapeDtypeStruct((B, 1, 128), jnp.int32),
        ],
        scratch_shapes=[pltpu.VMEM((K, 128), jnp.float32)],
    )(cand, rids)


def _rowmax_kernel(x_ref, o_ref):
    i = pl.program_id(1)
    rm = jnp.max(x_ref[...], axis=-1)
    n_of = jax.lax.broadcasted_iota(jnp.int32, (1, ROW_BLK), 1) + i * ROW_BLK
    o_ref[0] = jnp.where(n_of < N, rm, NEG)


@jax.jit
def _rowmax(pred_logits):
    out = pl.pallas_call(
        _rowmax_kernel,
        grid=(B, NB),
        in_specs=[pl.BlockSpec((1, ROW_BLK, C), lambda b, i: (b, i, 0))],
        out_specs=pl.BlockSpec((1, 1, ROW_BLK), lambda b, i: (b * NB + i, 0, 0)),
        out_shape=jax.ShapeDtypeStruct((B * NB, 1, ROW_BLK), jnp.float32),
    )(pred_logits)
    return out.reshape(B, NB * ROW_BLK)


@jax.jit
def kernel(pred_logits, pred_boxes, target_sizes):
    rowmax = _rowmax(pred_logits)
    _, rids_t = jax.lax.top_k(rowmax, K)
    rids = rids_t[:, None, :]
    rids = jnp.pad(rids, ((0, 0), (0, 0), (0, 128 - K)))
    rids100 = jnp.sort(rids[:, 0, :K], axis=1)
    cand = jnp.take_along_axis(pred_logits, rids100[:, :, None], axis=1)
    rids_pad = jnp.pad(rids100, ((0, 0), (0, 128 - K)))[:, None, :]
    vals, labs, ns = _final_topk(cand, rids_pad)
    scores = vals[:, 0, :K]
    labels = labs[:, 0, :K]
    topk_boxes = ns[:, 0, :K]

    gather_idx = jnp.broadcast_to(topk_boxes[:, :, None], (B, K, 4))
    bx = jnp.take_along_axis(pred_boxes, gather_idx, axis=1)
    cx, cy, w, h = jnp.split(bx, 4, axis=-1)
    boxes = jnp.concatenate(
        [cx - 0.5 * w, cy - 0.5 * h, cx + 0.5 * w, cy + 0.5 * h], axis=-1
    )
    img_h = target_sizes[:, 0]
    img_w = target_sizes[:, 1]
    scale_fct = jnp.stack([img_w, img_h, img_w, img_h], axis=1)
    boxes = boxes * scale_fct[:, None, :]
    return scores, labels, boxes


# speedup vs baseline: 3.4338x; 3.3133x over previous
import jax
import jax.numpy as jnp
from jax.experimental import pallas as pl
from jax.experimental.pallas import tpu as pltpu

B, N, C = 16, 20000, 91
K = 100
ROW_BLK = 2048
NB = (N + ROW_BLK - 1) // ROW_BLK
NROWS = NB * ROW_BLK // 128
NEG = float("-inf")
BIG = 2**30


def _select_rows_kernel(x_ref, ns_ref, rm_scratch):
    i = pl.program_id(1)
    rm = jnp.max(x_ref[...], axis=-1)
    n_of = jax.lax.broadcasted_iota(jnp.int32, (1, ROW_BLK), 1) + i * ROW_BLK
    rm = jnp.where(n_of < N, rm, NEG)
    rm_scratch[pl.ds(i * (ROW_BLK // 128), ROW_BLK // 128), :] = rm.reshape(
        ROW_BLK // 128, 128
    )

    @pl.when(i == NB - 1)
    def _():
        iota_r = jax.lax.broadcasted_iota(jnp.int32, (NROWS, 1), 0)
        iota_l = jax.lax.broadcasted_iota(jnp.int32, (1, 128), 1)
        rsum = jnp.max(rm_scratch[...], axis=1, keepdims=True)

        def body(it, carry):
            rsum, ns = carry
            m = jnp.max(rsum)
            rj = jnp.min(jnp.where(rsum == m, iota_r, BIG))
            row = rm_scratch[pl.ds(rj, 1), :]
            c = jnp.min(jnp.where(row == m, iota_l, BIG))
            n = rj * 128 + c
            new_row = jnp.where(iota_l == c, NEG, row)
            rm_scratch[pl.ds(rj, 1), :] = new_row
            rsum = jnp.where(iota_r == rj, jnp.max(new_row), rsum)
            ns = jnp.where(iota_l == it, n, ns)
            return rsum, ns

        _, ns = jax.lax.fori_loop(
            0, K, body, (rsum, jnp.zeros((1, 128), jnp.int32))
        )
        ns_ref[0, :, :] = ns


@jax.jit
def _select_rows(pred_logits):
    return pl.pallas_call(
        _select_rows_kernel,
        grid=(B, NB),
        in_specs=[pl.BlockSpec((1, ROW_BLK, C), lambda b, i: (b, i, 0))],
        out_specs=pl.BlockSpec((1, 1, 128), lambda b, i: (b, 0, 0)),
        out_shape=jax.ShapeDtypeStruct((B, 1, 128), jnp.int32),
        scratch_shapes=[pltpu.VMEM((NROWS, 128), jnp.float32)],
    )(pred_logits)


FLAT = K * C
FLATP = 9216


def _final_topk_kernel(x_ref, g_ref, val_ref, lab_ref, ns_ref, ds):
    iota_l = jax.lax.broadcasted_iota(jnp.int32, (B, 128), 1)
    ds[...] = x_ref[...]
    gf = g_ref[...]

    def body(it, carry):
        vals, gs = carry
        m = jnp.max(ds[...], axis=1, keepdims=True)
        g = jnp.min(jnp.where(ds[...] == m, gf, BIG), axis=1, keepdims=True)
        ds[...] = jnp.where(gf == g, NEG, ds[...])
        vals = jnp.where(iota_l == it, m, vals)
        gs = jnp.where(iota_l == it, g, gs)
        return vals, gs

    zf = jnp.zeros((B, 128), jnp.float32)
    zi = jnp.zeros((B, 128), jnp.int32)
    vals, gs = jax.lax.fori_loop(0, K, body, (zf, zi))
    val_ref[...] = jax.nn.sigmoid(vals)
    lab_ref[...] = gs % C
    ns_ref[...] = gs // C


@jax.jit
def _final_topk(candp, gflatp):
    return pl.pallas_call(
        _final_topk_kernel,
        out_shape=[
            jax.ShapeDtypeStruct((B, 128), jnp.float32),
            jax.ShapeDtypeStruct((B, 128), jnp.int32),
            jax.ShapeDtypeStruct((B, 128), jnp.int32),
        ],
        scratch_shapes=[pltpu.VMEM((B, FLATP), jnp.float32)],
    )(candp, gflatp)


def _rowmax_kernel(x_ref, o_ref):
    i = pl.program_id(1)
    rm = jnp.max(x_ref[...], axis=-1)
    n_of = jax.lax.broadcasted_iota(jnp.int32, (1, ROW_BLK), 1) + i * ROW_BLK
    o_ref[0] = jnp.where(n_of < N, rm, NEG)


@jax.jit
def _rowmax(pred_logits):
    out = pl.pallas_call(
        _rowmax_kernel,
        grid=(B, NB),
        in_specs=[pl.BlockSpec((1, ROW_BLK, C), lambda b, i: (b, i, 0))],
        out_specs=pl.BlockSpec((1, 1, ROW_BLK), lambda b, i: (b * NB + i, 0, 0)),
        out_shape=jax.ShapeDtypeStruct((B * NB, 1, ROW_BLK), jnp.float32),
    )(pred_logits)
    return out.reshape(B, NB * ROW_BLK)


@jax.jit
def kernel(pred_logits, pred_boxes, target_sizes):
    rowmax = _rowmax(pred_logits)
    _, rids100 = jax.lax.top_k(rowmax, K)
    cand = jnp.take_along_axis(pred_logits, rids100[:, :, None], axis=1)
    gflat = rids100[:, :, None] * C + jnp.arange(C, dtype=jnp.int32)
    candp = jnp.pad(cand.reshape(B, FLAT), ((0, 0), (0, FLATP - FLAT)),
                    constant_values=NEG)
    gflatp = jnp.pad(gflat.reshape(B, FLAT), ((0, 0), (0, FLATP - FLAT)),
                     constant_values=BIG)
    vals, labs, ns = _final_topk(candp, gflatp)
    scores = vals[:, :K]
    labels = labs[:, :K]
    topk_boxes = ns[:, :K]

    gather_idx = jnp.broadcast_to(topk_boxes[:, :, None], (B, K, 4))
    bx = jnp.take_along_axis(pred_boxes, gather_idx, axis=1)
    cx, cy, w, h = jnp.split(bx, 4, axis=-1)
    boxes = jnp.concatenate(
        [cx - 0.5 * w, cy - 0.5 * h, cx + 0.5 * w, cy + 0.5 * h], axis=-1
    )
    img_h = target_sizes[:, 0]
    img_w = target_sizes[:, 1]
    scale_fct = jnp.stack([img_w, img_h, img_w, img_h], axis=1)
    boxes = boxes * scale_fct[:, None, :]
    return scores, labels, boxes
